# baseline (speedup 1.0000x reference)
import jax
import jax.numpy as jnp
from jax.experimental import pallas as pl
from jax.experimental.pallas import tpu as pltpu


def _reduce_block(a, ys3, xs):
    colsum = a.sum(axis=1)
    wsum = (a * ys3).sum(axis=1)
    zeta = colsum.sum(axis=1)
    kx = jnp.round((colsum * xs).sum(axis=1) / zeta)
    ky = jnp.round(wsum.sum(axis=1) / zeta)
    return zeta, kx, ky


def _kernel(hm_ref, tf_ref, map_ref, zeta_ref, kp_ref, tf_kp_ref):
    C, H, W = map_ref.shape[1:]
    ys3 = jax.lax.broadcasted_iota(jnp.int32, (1, H, W), 1).astype(jnp.float32)
    xs = jax.lax.broadcasted_iota(jnp.int32, (C, W), 1).astype(jnp.float32)

    a = jnp.abs(hm_ref[0])
    map_ref[0] = a
    zeta, kx, ky = _reduce_block(a, ys3, xs)
    zeta_ref[0, 0, :] = zeta
    kp_ref[0] = jnp.stack([kx, ky], axis=-1)

    t = jnp.abs(tf_ref[0])
    _, tkx, tky = _reduce_block(t, ys3, xs)
    tf_kp_ref[0] = jnp.stack([tkx, tky], axis=-1)


def kernel(combined_hm_preds, tf_combined_hm_preds, cur_batch):
    B, C, H, W = combined_hm_preds.shape
    in_spec = pl.BlockSpec((1, C, H, W), lambda b: (b, 0, 0, 0))
    map_val, zeta, kp, tf_kp = pl.pallas_call(
        _kernel,
        grid=(B,),
        in_specs=[in_spec, in_spec],
        out_specs=(
            pl.BlockSpec((1, C, H, W), lambda b: (b, 0, 0, 0)),
            pl.BlockSpec((1, 1, C), lambda b: (b, 0, 0)),
            pl.BlockSpec((1, C, 2), lambda b: (b, 0, 0)),
            pl.BlockSpec((1, C, 2), lambda b: (b, 0, 0)),
        ),
        out_shape=(
            jax.ShapeDtypeStruct((B, C, H, W), jnp.float32),
            jax.ShapeDtypeStruct((B, 1, C), jnp.float32),
            jax.ShapeDtypeStruct((B, C, 2), jnp.float32),
            jax.ShapeDtypeStruct((B, C, 2), jnp.float32),
        ),
        compiler_params=pltpu.CompilerParams(
            dimension_semantics=("parallel",),
            vmem_limit_bytes=56 * 1024 * 1024,
        ),
    )(combined_hm_preds, tf_combined_hm_preds)
    return (map_val, kp, zeta.reshape(B, C), tf_kp)


# speedup vs baseline: 1.0054x; 1.0054x over previous
import jax
import jax.numpy as jnp
from jax.experimental import pallas as pl
from jax.experimental.pallas import tpu as pltpu


def _reduce_block(a, ys3, xs):
    colsum = a.sum(axis=1)
    wsum = (a * ys3).sum(axis=1)
    zeta = colsum.sum(axis=1)
    kx = jnp.round((colsum * xs).sum(axis=1) / zeta)
    ky = jnp.round(wsum.sum(axis=1) / zeta)
    return zeta, kx, ky


def _kernel(hm_ref, tf_ref, map_ref, small_ref):
    C, H, W = map_ref.shape[1:]
    i = pl.program_id(1)
    ys3 = jax.lax.broadcasted_iota(jnp.int32, (1, H, W), 1).astype(jnp.float32)
    xs = jax.lax.broadcasted_iota(jnp.int32, (C, W), 1).astype(jnp.float32)

    a = jnp.abs(hm_ref[0])
    map_ref[0] = a
    zeta, kx, ky = _reduce_block(a, ys3, xs)

    t = jnp.abs(tf_ref[0])
    _, tkx, tky = _reduce_block(t, ys3, xs)

    small_ref[i] = jnp.stack([zeta, kx, ky, tkx, tky, zeta, zeta, zeta], axis=0)


def kernel(combined_hm_preds, tf_combined_hm_preds, cur_batch):
    B, C, H, W = combined_hm_preds.shape
    half = B // 2
    in_spec = pl.BlockSpec((1, C, H, W), lambda c, i: (c * half + i, 0, 0, 0))
    map_val, small = pl.pallas_call(
        _kernel,
        grid=(2, half),
        in_specs=[in_spec, in_spec],
        out_specs=(
            pl.BlockSpec((1, C, H, W), lambda c, i: (c * half + i, 0, 0, 0)),
            pl.BlockSpec((half, 8, C), lambda c, i: (c, 0, 0)),
        ),
        out_shape=(
            jax.ShapeDtypeStruct((B, C, H, W), jnp.float32),
            jax.ShapeDtypeStruct((B, 8, C), jnp.float32),
        ),
        compiler_params=pltpu.CompilerParams(
            dimension_semantics=("parallel", "arbitrary"),
            vmem_limit_bytes=56 * 1024 * 1024,
        ),
    )(combined_hm_preds, tf_combined_hm_preds)
    zeta = small[:, 0, :]
    keypoint = jnp.stack([small[:, 1, :], small[:, 2, :]], axis=-1)
    tf_keypoint = jnp.stack([small[:, 3, :], small[:, 4, :]], axis=-1)
    return (map_val, keypoint, zeta, tf_keypoint)


# speedup vs baseline: 1.0112x; 1.0057x over previous
import jax
import jax.numpy as jnp
from jax.experimental import pallas as pl
from jax.experimental.pallas import tpu as pltpu


def _reduce_block(a, ys3, xs):
    colsum = a.sum(axis=1)
    wsum = (a * ys3).sum(axis=1)
    zeta = colsum.sum(axis=1)
    kx = jnp.round((colsum * xs).sum(axis=1) / zeta)
    ky = jnp.round(wsum.sum(axis=1) / zeta)
    return zeta, kx, ky


def _kernel(hm_ref, tf_ref, map_ref, small_ref):
    C, H, W = map_ref.shape[1:]
    ys3 = jax.lax.broadcasted_iota(jnp.int32, (1, H, W), 1).astype(jnp.float32)
    xs = jax.lax.broadcasted_iota(jnp.int32, (C, W), 1).astype(jnp.float32)

    a = jnp.abs(hm_ref[0])
    map_ref[0] = a
    zeta, kx, ky = _reduce_block(a, ys3, xs)

    t = jnp.abs(tf_ref[0])
    _, tkx, tky = _reduce_block(t, ys3, xs)

    small_ref[0] = jnp.stack([zeta, kx, ky, tkx, tky, zeta, zeta, zeta], axis=0)


def kernel(combined_hm_preds, tf_combined_hm_preds, cur_batch):
    B, C, H, W = combined_hm_preds.shape
    in_spec = pl.BlockSpec((1, C, H, W), lambda b: (b, 0, 0, 0))
    map_val, small = pl.pallas_call(
        _kernel,
        grid=(B,),
        in_specs=[in_spec, in_spec],
        out_specs=(
            pl.BlockSpec((1, C, H, W), lambda b: (b, 0, 0, 0)),
            pl.BlockSpec((1, 8, C), lambda b: (b, 0, 0)),
        ),
        out_shape=(
            jax.ShapeDtypeStruct((B, C, H, W), jnp.float32),
            jax.ShapeDtypeStruct((B, 8, C), jnp.float32),
        ),
        compiler_params=pltpu.CompilerParams(
            dimension_semantics=("parallel",),
            vmem_limit_bytes=56 * 1024 * 1024,
        ),
    )(combined_hm_preds, tf_combined_hm_preds)
    zeta = small[:, 0, :]
    keypoint = jnp.stack([small[:, 1, :], small[:, 2, :]], axis=-1)
    tf_keypoint = jnp.stack([small[:, 3, :], small[:, 4, :]], axis=-1)
    return (map_val, keypoint, zeta, tf_keypoint)
